# baseline (speedup 1.0000x reference)
import functools

import numpy as np
import jax
import jax.numpy as jnp
from jax import lax
from jax.experimental import pallas as pl
from jax.experimental.pallas import tpu as pltpu
from jax.experimental.pallas import tpu_sc as plsc

_N_LEVELS = 16
_F = 2
_T = 2 ** 19
_BASE_RES = 16
_PER_LEVEL_SCALE = 1.5
_HIDDEN = 64

_NC = 2
_NS = 16
_LANES = 16
_NW = _NC * _NS

_P1 = np.int32(-1640531535)
_MASK = np.int32(_T - 1)

_C = 128
_LC = _N_LEVELS * 4


def _enc_body(x_hbm, y_hbm, tab_hbm, out_hbm, xv, yv, idxv, wv, rowsv, encv, sem):
    wid = lax.axis_index("s") * _NC + lax.axis_index("c")
    n = out_hbm.shape[1]
    ppw = n // _NW
    nchunk = ppw // _C
    iota = lax.iota(jnp.int32, _LANES)

    def chunk(t, carry):
        base = wid * ppw + t * _C
        pltpu.sync_copy(x_hbm.at[pl.ds(base, _C)], xv)
        pltpu.sync_copy(y_hbm.at[pl.ds(base, _C)], yv)

        def pass_a(i, c):
            off = i * _LANES
            sl = pl.ds(off, _LANES)
            xb = xv[sl]
            yb = yv[sl]
            for l in range(_N_LEVELS):
                s = np.float32(_BASE_RES * _PER_LEVEL_SCALE ** l)
                px = xb * s
                py = yb * s
                ix = px.astype(jnp.int32)
                iy = py.astype(jnp.int32)
                wx = px - ix.astype(jnp.float32)
                wy = py - iy.astype(jnp.float32)
                t0 = iy * _P1
                t1 = t0 + _P1
                ix1 = ix + 1
                lb = jnp.int32(l * _T)
                idxv[4 * l + 0, sl] = ((ix ^ t0) & _MASK) | lb
                idxv[4 * l + 1, sl] = ((ix1 ^ t0) & _MASK) | lb
                idxv[4 * l + 2, sl] = ((ix ^ t1) & _MASK) | lb
                idxv[4 * l + 3, sl] = ((ix1 ^ t1) & _MASK) | lb
                ox = 1.0 - wx
                oy = 1.0 - wy
                wv[4 * l + 0, sl] = ox * oy
                wv[4 * l + 1, sl] = wx * oy
                wv[4 * l + 2, sl] = ox * wy
                wv[4 * l + 3, sl] = wx * wy
            return c

        lax.fori_loop(0, _C // _LANES, pass_a, 0, unroll=False)

        for lc in range(_LC):
            pltpu.async_copy(tab_hbm.at[idxv.at[lc]], rowsv.at[pl.ds(lc * _C, _C)],
                             sem.at[0]).wait()

        def pass_b(i, c):
            off = i * _LANES
            sl = pl.ds(off, _LANES)
            pti = off + iota
            for l in range(_N_LEVELS):
                w00 = wv[4 * l + 0, sl]
                w10 = wv[4 * l + 1, sl]
                w01 = wv[4 * l + 2, sl]
                w11 = wv[4 * l + 3, sl]
                acc = [None, None]
                for f in range(_F):
                    fsp = jnp.full((_LANES,), f, jnp.int32)
                    c00 = plsc.load_gather(rowsv, [(4 * l + 0) * _C + pti, fsp])
                    c10 = plsc.load_gather(rowsv, [(4 * l + 1) * _C + pti, fsp])
                    c01 = plsc.load_gather(rowsv, [(4 * l + 2) * _C + pti, fsp])
                    c11 = plsc.load_gather(rowsv, [(4 * l + 3) * _C + pti, fsp])
                    acc[f] = (w00 * c00 + w10 * c10) + (w01 * c01 + w11 * c11)
                encv[2 * l + 0, sl] = acc[0]
                encv[2 * l + 1, sl] = acc[1]
            return c

        lax.fori_loop(0, _C // _LANES, pass_b, 0, unroll=False)
        pltpu.sync_copy(encv, out_hbm.at[:, pl.ds(base, _C)])
        return carry

    lax.fori_loop(0, nchunk, chunk, 0, unroll=False)


def _encode_sc(x, y, tab_flat, n):
    mesh = plsc.VectorSubcoreMesh(core_axis_name="c", subcore_axis_name="s")
    f = pl.kernel(
        _enc_body,
        out_type=jax.ShapeDtypeStruct((2 * _N_LEVELS, n), jnp.float32),
        mesh=mesh,
        compiler_params=pltpu.CompilerParams(needs_layout_passes=False,
                                             use_tc_tiling_on_sc=False),
        scratch_types=[
            pltpu.VMEM((_C,), jnp.float32),
            pltpu.VMEM((_C,), jnp.float32),
            pltpu.VMEM((_LC, _C), jnp.int32),
            pltpu.VMEM((_LC, _C), jnp.float32),
            pltpu.VMEM((_LC * _C, _F), jnp.float32),
            pltpu.VMEM((2 * _N_LEVELS, _C), jnp.float32),
            pltpu.SemaphoreType.DMA((8,)),
        ],
    )
    return f(x, y, tab_flat)


def _mlp_body(x_ref, w0_ref, w1_ref, w2_ref, o_ref):
    x = x_ref[...]
    h = jax.lax.dot_general(w0_ref[...], x, (((1,), (0,)), ((), ())),
                            precision=jax.lax.Precision.HIGHEST,
                            preferred_element_type=jnp.float32)
    h = jnp.maximum(h, 0.0)
    h = jax.lax.dot_general(w1_ref[...], h, (((1,), (0,)), ((), ())),
                            precision=jax.lax.Precision.HIGHEST,
                            preferred_element_type=jnp.float32)
    h = jnp.maximum(h, 0.0)
    o_ref[...] = jax.lax.dot_general(w2_ref[...], h, (((1,), (0,)), ((), ())),
                                     precision=jax.lax.Precision.HIGHEST,
                                     preferred_element_type=jnp.float32)


def _mlp_tc(enc, w0t, w1t, w2t, n):
    bn = 4096
    grid = n // bn
    in_enc = 2 * _N_LEVELS
    return pl.pallas_call(
        _mlp_body,
        grid=(grid,),
        in_specs=[
            pl.BlockSpec((in_enc, bn), lambda j: (0, j)),
            pl.BlockSpec((_HIDDEN, in_enc), lambda j: (0, 0)),
            pl.BlockSpec((_HIDDEN, _HIDDEN), lambda j: (0, 0)),
            pl.BlockSpec((8, _HIDDEN), lambda j: (0, 0)),
        ],
        out_specs=pl.BlockSpec((8, bn), lambda j: (0, j)),
        out_shape=jax.ShapeDtypeStruct((8, n), jnp.float32),
    )(enc, w0t, w1t, w2t)


def kernel(xy, tables, W0, W1, W2):
    n = xy.shape[0]
    res = int(round(np.sqrt(n)))
    x = xy[:, 0]
    y = xy[:, 1]
    tab_flat = tables.reshape(_N_LEVELS * _T, _F)
    enc = _encode_sc(x, y, tab_flat, n)
    w0t = W0.T
    w1t = W1.T
    w2t = jnp.zeros((8, _HIDDEN), jnp.float32).at[:W2.shape[1]].set(W2.T)
    out = _mlp_tc(enc, w0t, w1t, w2t, n)
    return out[:3].reshape(3, res, res)


# speedup vs baseline: 2.1814x; 2.1814x over previous
import numpy as np
import jax
import jax.numpy as jnp
from jax import lax
from jax.experimental import pallas as pl
from jax.experimental.pallas import tpu as pltpu
from jax.experimental.pallas import tpu_sc as plsc

_N_LEVELS = 16
_F = 2
_T = 2 ** 19
_BASE_RES = 16
_PER_LEVEL_SCALE = 1.5
_HIDDEN = 64

_NC = 2
_NS = 16
_LANES = 16
_NW = _NC * _NS

_P1 = np.int32(-1640531535)
_MASK = np.int32(_T - 1)
_HI = np.int32(_T - 1 - 127)
_LO = np.int32(127)

_C = 128
_NS_STREAMS = _N_LEVELS * 4 * _F


def _enc_body(x_hbm, y_hbm, tab_hbm, out_hbm, xv, yv, idxv, wv, rowsv, encv, sem):
    wid = lax.axis_index("s") * _NC + lax.axis_index("c")
    n = out_hbm.shape[1]
    ppw = n // _NW
    nchunk = ppw // _C
    iota = lax.iota(jnp.int32, _LANES)

    def chunk(t, carry):
        base = wid * ppw + t * _C
        pltpu.sync_copy(x_hbm.at[pl.ds(base, _C)], xv)
        pltpu.sync_copy(y_hbm.at[pl.ds(base, _C)], yv)

        def pass_a(i, c):
            off = i * _LANES
            sl = pl.ds(off, _LANES)
            xb = xv[sl]
            yb = yv[sl]
            for l in range(_N_LEVELS):
                s = np.float32(_BASE_RES * _PER_LEVEL_SCALE ** l)
                px = xb * s
                py = yb * s
                ix = px.astype(jnp.int32)
                iy = py.astype(jnp.int32)
                wx = px - ix.astype(jnp.float32)
                wy = py - iy.astype(jnp.float32)
                t0 = iy * _P1
                t1 = t0 + _P1
                ix1 = ix + 1
                lb = jnp.int32(l << 20)
                for c4, h in enumerate((ix ^ t0, ix1 ^ t0, ix ^ t1, ix1 ^ t1)):
                    u = (((h & _HI) << 1) | (h & _LO)) | lb
                    idxv[8 * l + 2 * c4, sl] = u
                    idxv[8 * l + 2 * c4 + 1, sl] = u + 128
                ox = 1.0 - wx
                oy = 1.0 - wy
                wv[4 * l + 0, sl] = ox * oy
                wv[4 * l + 1, sl] = wx * oy
                wv[4 * l + 2, sl] = ox * wy
                wv[4 * l + 3, sl] = wx * wy
            return c

        lax.fori_loop(0, _C // _LANES, pass_a, 0, unroll=False)

        for k in range(_NS_STREAMS):
            pltpu.async_copy(tab_hbm.at[idxv.at[k]], rowsv.at[pl.ds(k * _C, _C)],
                             sem.at[0]).wait()

        def pass_b(i, c):
            off = i * _LANES
            sl = pl.ds(off, _LANES)
            for l in range(_N_LEVELS):
                acc = [None, None]
                for f in range(_F):
                    a = None
                    for c4 in range(4):
                        w = wv[4 * l + c4, sl]
                        v = rowsv[pl.ds((8 * l + 2 * c4 + f) * _C + off, _LANES)]
                        a = w * v if a is None else a + w * v
                    acc[f] = a
                encv[2 * l + 0, sl] = acc[0]
                encv[2 * l + 1, sl] = acc[1]
            return c

        lax.fori_loop(0, _C // _LANES, pass_b, 0, unroll=False)
        pltpu.sync_copy(encv, out_hbm.at[:, pl.ds(base, _C)])
        return carry

    lax.fori_loop(0, nchunk, chunk, 0, unroll=False)


def _encode_sc(x, y, tab_words, n):
    mesh = plsc.VectorSubcoreMesh(core_axis_name="c", subcore_axis_name="s")
    f = pl.kernel(
        _enc_body,
        out_type=jax.ShapeDtypeStruct((2 * _N_LEVELS, n), jnp.float32),
        mesh=mesh,
        compiler_params=pltpu.CompilerParams(needs_layout_passes=False,
                                             use_tc_tiling_on_sc=False),
        scratch_types=[
            pltpu.VMEM((_C,), jnp.float32),
            pltpu.VMEM((_C,), jnp.float32),
            pltpu.VMEM((_NS_STREAMS, _C), jnp.int32),
            pltpu.VMEM((_N_LEVELS * 4, _C), jnp.float32),
            pltpu.VMEM((_NS_STREAMS * _C,), jnp.float32),
            pltpu.VMEM((2 * _N_LEVELS, _C), jnp.float32),
            pltpu.SemaphoreType.DMA((8,)),
        ],
    )
    return f(x, y, tab_words)


def _mlp_body(x_ref, w0_ref, w1_ref, w2_ref, o_ref):
    x = x_ref[...]
    h = jax.lax.dot_general(w0_ref[...], x, (((1,), (0,)), ((), ())),
                            precision=jax.lax.Precision.HIGHEST,
                            preferred_element_type=jnp.float32)
    h = jnp.maximum(h, 0.0)
    h = jax.lax.dot_general(w1_ref[...], h, (((1,), (0,)), ((), ())),
                            precision=jax.lax.Precision.HIGHEST,
                            preferred_element_type=jnp.float32)
    h = jnp.maximum(h, 0.0)
    o_ref[...] = jax.lax.dot_general(w2_ref[...], h, (((1,), (0,)), ((), ())),
                                     precision=jax.lax.Precision.HIGHEST,
                                     preferred_element_type=jnp.float32)


def _mlp_tc(enc, w0t, w1t, w2t, n):
    bn = 4096
    grid = n // bn
    in_enc = 2 * _N_LEVELS
    return pl.pallas_call(
        _mlp_body,
        grid=(grid,),
        in_specs=[
            pl.BlockSpec((in_enc, bn), lambda j: (0, j)),
            pl.BlockSpec((_HIDDEN, in_enc), lambda j: (0, 0)),
            pl.BlockSpec((_HIDDEN, _HIDDEN), lambda j: (0, 0)),
            pl.BlockSpec((8, _HIDDEN), lambda j: (0, 0)),
        ],
        out_specs=pl.BlockSpec((8, bn), lambda j: (0, j)),
        out_shape=jax.ShapeDtypeStruct((8, n), jnp.float32),
    )(enc, w0t, w1t, w2t)


def kernel(xy, tables, W0, W1, W2):
    n = xy.shape[0]
    res = int(round(np.sqrt(n)))
    x = xy[:, 0]
    y = xy[:, 1]
    one = xy[0, 0] * 0.0 + 1.0
    tab_words = (tables.reshape(_N_LEVELS, _T // _C, _C, _F)
                 .transpose(0, 1, 3, 2)
                 .reshape(_N_LEVELS * _T * _F) * one)
    enc = _encode_sc(x, y, tab_words, n)
    w0t = W0.T
    w1t = W1.T
    w2t = jnp.zeros((8, _HIDDEN), jnp.float32).at[:W2.shape[1]].set(W2.T)
    out = _mlp_tc(enc, w0t, w1t, w2t, n)
    return out[:3].reshape(3, res, res)
